# Initial kernel scaffold; baseline (speedup 1.0000x reference)
import jax
import jax.numpy as jnp
from jax.experimental import pallas as pl


def kernel(x, edge_index, label, eps_pos, eps_neg, weight_self, weight_pos, weight_neg):
    raise NotImplementedError("write your pallas kernel here")


# speedup vs baseline: 10.7929x; 10.7929x over previous
import functools

import jax
import jax.numpy as jnp
from jax import lax
from jax.experimental import pallas as pl
from jax.experimental.pallas import tpu as pltpu
from jax.experimental.pallas import tpu_sc as plsc

N_NODES = 10000
N_EDGES = 320000
D = 128
H = 64
NC = 2
NS = 16
L = 16
CHUNK = 128
EPT = 20096
CHUNKS = EPT // CHUNK
PAD_E = EPT * NS
DUMMY = N_NODES
ACC_ROWS = 10240
ZROWS = 128
ROWS_PER_TILE_OUT = N_NODES // NS


def _sc_body(x2, edges, lbl, out_a, out_b,
             ebuf, gbuf, abuf, rows, lbuf, zbuf, acc_a, acc_b):
    cid = lax.axis_index("c")
    sid = lax.axis_index("s")

    zero = jnp.zeros((L,), jnp.float32)
    for r in range(ZROWS):
        for c in range(H // L):
            zbuf[r, pl.ds(c * L, L)] = zero
    slab = ACC_ROWS // NS
    for k in range(slab // ZROWS):
        pltpu.sync_copy(zbuf, acc_a.at[pl.ds(sid * slab + k * ZROWS, ZROWS)])
        pltpu.sync_copy(zbuf, acc_b.at[pl.ds(sid * slab + k * ZROWS, ZROWS)])

    pltpu.sync_copy(lbl, lbuf)

    plsc.subcore_barrier()

    def chunk_body(j, carry):
        pltpu.sync_copy(edges.at[sid, j], ebuf)
        for g in range(CHUNK // L):
            s = ebuf[0, pl.ds(g * L, L)]
            d = ebuf[1, pl.ds(g * L, L)]
            lv = plsc.load_gather(lbuf, [s])
            gbuf[pl.ds(g * L, L)] = s * 2 + cid
            abuf[pl.ds(g * L, L)] = jnp.where(lv == 1, d, DUMMY)
        pltpu.sync_copy(x2.at[gbuf], rows)
        pltpu.sync_copy(rows, acc_b.at[ebuf.at[1]], add=True)
        pltpu.sync_copy(rows, acc_a.at[abuf], add=True)
        return carry

    lax.fori_loop(0, CHUNKS, chunk_body, 0)

    plsc.subcore_barrier()

    r0 = sid * (ACC_ROWS // NS)
    pltpu.sync_copy(acc_a.at[pl.ds(r0, ACC_ROWS // NS)],
                    out_a.at[cid, pl.ds(r0, ACC_ROWS // NS)])
    pltpu.sync_copy(acc_b.at[pl.ds(r0, ACC_ROWS // NS)],
                    out_b.at[cid, pl.ds(r0, ACC_ROWS // NS)])


_sc_scatter = functools.partial(
    pl.kernel,
    out_type=[
        jax.ShapeDtypeStruct((NC, ACC_ROWS, H), jnp.float32),
        jax.ShapeDtypeStruct((NC, ACC_ROWS, H), jnp.float32),
    ],
    mesh=plsc.VectorSubcoreMesh(core_axis_name="c", subcore_axis_name="s"),
    compiler_params=pltpu.CompilerParams(needs_layout_passes=False,
                                         use_tc_tiling_on_sc=False),
    scratch_types=[
        pltpu.VMEM((2, CHUNK), jnp.int32),
        pltpu.VMEM((CHUNK,), jnp.int32),
        pltpu.VMEM((CHUNK,), jnp.int32),
        pltpu.VMEM((CHUNK, H), jnp.float32),
        pltpu.VMEM((N_NODES,), jnp.int32),
        pltpu.VMEM((ZROWS, H), jnp.float32),
        pltpu.VMEM_SHARED((ACC_ROWS, H), jnp.float32),
        pltpu.VMEM_SHARED((ACC_ROWS, H), jnp.float32),
    ],
)(_sc_body)


BM = 1000


def _tc_body(ep_ref, en_ref, x_ref, a_ref, b_ref, ws_ref, wp_ref, wn_ref,
             out_ref):
    t = jnp.concatenate([a_ref[0], a_ref[1]], axis=1)
    s = jnp.concatenate([b_ref[0], b_ref[1]], axis=1)
    hp = s + t
    ep = ep_ref[0, 0]
    en = en_ref[0, 0]
    hf = x_ref[...] + (1.0 + ep) * hp + (1.0 + en) * t
    o1 = jnp.dot(hf, ws_ref[...], preferred_element_type=jnp.float32)
    o2 = jnp.dot(hp, wp_ref[...], preferred_element_type=jnp.float32)
    o3 = jnp.dot(t, wn_ref[...], preferred_element_type=jnp.float32)
    out_ref[...] = jnp.maximum(jnp.concatenate([o1, o2, o3], axis=1), 0.0)


_tc_finish = pl.pallas_call(
    _tc_body,
    grid=(N_NODES // BM,),
    in_specs=[
        pl.BlockSpec(memory_space=pltpu.SMEM),
        pl.BlockSpec(memory_space=pltpu.SMEM),
        pl.BlockSpec((BM, D), lambda i: (i, 0)),
        pl.BlockSpec((NC, BM, H), lambda i: (0, i, 0)),
        pl.BlockSpec((NC, BM, H), lambda i: (0, i, 0)),
        pl.BlockSpec((D, D), lambda i: (0, 0)),
        pl.BlockSpec((D, D), lambda i: (0, 0)),
        pl.BlockSpec((D, D), lambda i: (0, 0)),
    ],
    out_specs=pl.BlockSpec((BM, 3 * D), lambda i: (i, 0)),
    out_shape=jax.ShapeDtypeStruct((N_NODES, 3 * D), jnp.float32),
)


@jax.jit
def kernel(x, edge_index, label, eps_pos, eps_neg,
           weight_self, weight_pos, weight_neg):
    src = edge_index[0].astype(jnp.int32)
    dst = edge_index[1].astype(jnp.int32)
    lbl = label.astype(jnp.int32)
    x2 = x.reshape(2 * N_NODES, H)

    pad = PAD_E - N_EDGES
    src_p = jnp.concatenate([src, jnp.zeros((pad,), jnp.int32)])
    dst_p = jnp.concatenate([dst, jnp.full((pad,), DUMMY, jnp.int32)])
    edges = jnp.stack([src_p.reshape(NS, CHUNKS, CHUNK),
                       dst_p.reshape(NS, CHUNKS, CHUNK)], axis=2)

    t_half, s_half = _sc_scatter(x2, edges, lbl)

    return _tc_finish(eps_pos.reshape(1, 1), eps_neg.reshape(1, 1),
                      x, t_half, s_half,
                      weight_self, weight_pos, weight_neg)


# speedup vs baseline: 11.4700x; 1.0627x over previous
import functools

import jax
import jax.numpy as jnp
from jax import lax
from jax.experimental import pallas as pl
from jax.experimental.pallas import tpu as pltpu
from jax.experimental.pallas import tpu_sc as plsc

N_NODES = 10000
N_EDGES = 320000
D = 128
H = 64
NC = 2
NS = 16
L = 16
CHUNK = 128
CHUNKS = 158
ALLOC_CHUNKS = CHUNKS + 2
EPT = CHUNKS * CHUNK
PAD_E = EPT * NS
DUMMY = N_NODES
ACC_ROWS = 10240
ZROWS = 128
PAIRS = CHUNKS // 2


def _sc_body(x2, edges, lbl, out_a, out_b,
             ebuf, gbuf, abuf, dbuf, rows, lbuf, zbuf, acc_a, acc_b,
             sem_i, sem_g, sem_s):
    cid = lax.axis_index("c")
    sid = lax.axis_index("s")

    idx0 = pltpu.async_copy(edges.at[sid, 0], ebuf.at[0], sem_i.at[0])
    idx1 = pltpu.async_copy(edges.at[sid, 1], ebuf.at[1], sem_i.at[1])

    zero = jnp.zeros((L,), jnp.float32)
    for r in range(ZROWS):
        for c in range(H // L):
            zbuf[r, pl.ds(c * L, L)] = zero
    slab = ACC_ROWS // NS
    for k in range(slab // ZROWS):
        pltpu.sync_copy(zbuf, acc_a.at[pl.ds(sid * slab + k * ZROWS, ZROWS)])
        pltpu.sync_copy(zbuf, acc_b.at[pl.ds(sid * slab + k * ZROWS, ZROWS)])

    pltpu.sync_copy(lbl, lbuf)

    plsc.subcore_barrier()

    def stage(j, p, first):
        pltpu.make_async_copy(edges.at[sid, 0], ebuf.at[p], sem_i.at[p]).wait()
        if not first:
            for _ in range(2):
                pltpu.make_async_copy(
                    x2.at[pl.ds(0, CHUNK)], rows.at[p], sem_s.at[p]).wait()
        for g in range(CHUNK // L):
            s = ebuf[p, 0, pl.ds(g * L, L)]
            d = ebuf[p, 1, pl.ds(g * L, L)]
            lv = plsc.load_gather(lbuf, [s])
            gbuf[p, pl.ds(g * L, L)] = s * 2 + cid
            dbuf[p, pl.ds(g * L, L)] = d
            abuf[p, pl.ds(g * L, L)] = jnp.where(lv == 1, d, DUMMY)
        pltpu.async_copy(edges.at[sid, j + 2], ebuf.at[p], sem_i.at[p])
        return pltpu.async_copy(x2.at[gbuf.at[p]], rows.at[p], sem_g.at[p])

    def scatters(p):
        pltpu.async_copy(rows.at[p], acc_b.at[dbuf.at[p]], sem_s.at[p],
                         add=True)
        pltpu.async_copy(rows.at[p], acc_a.at[abuf.at[p]], sem_s.at[p],
                         add=True)

    def pair(i, first):
        g0 = stage(2 * i, 0, first)
        g1 = stage(2 * i + 1, 1, first)
        g0.wait()
        scatters(0)
        g1.wait()
        scatters(1)
        return 0

    pair(0, True)
    lax.fori_loop(1, PAIRS, lambda i, _: pair(i, False), 0)

    for p in range(2):
        for _ in range(2):
            pltpu.make_async_copy(
                x2.at[pl.ds(0, CHUNK)], rows.at[p], sem_s.at[p]).wait()
        pltpu.make_async_copy(edges.at[sid, 0], ebuf.at[p],
                              sem_i.at[p]).wait()

    plsc.subcore_barrier()

    r0 = sid * (ACC_ROWS // NS)
    pltpu.sync_copy(acc_a.at[pl.ds(r0, ACC_ROWS // NS)],
                    out_a.at[cid, pl.ds(r0, ACC_ROWS // NS)])
    pltpu.sync_copy(acc_b.at[pl.ds(r0, ACC_ROWS // NS)],
                    out_b.at[cid, pl.ds(r0, ACC_ROWS // NS)])


_sc_scatter = functools.partial(
    pl.kernel,
    out_type=[
        jax.ShapeDtypeStruct((NC, ACC_ROWS, H), jnp.float32),
        jax.ShapeDtypeStruct((NC, ACC_ROWS, H), jnp.float32),
    ],
    mesh=plsc.VectorSubcoreMesh(core_axis_name="c", subcore_axis_name="s"),
    compiler_params=pltpu.CompilerParams(needs_layout_passes=False,
                                         use_tc_tiling_on_sc=False),
    scratch_types=[
        pltpu.VMEM((2, 2, CHUNK), jnp.int32),
        pltpu.VMEM((2, CHUNK), jnp.int32),
        pltpu.VMEM((2, CHUNK), jnp.int32),
        pltpu.VMEM((2, CHUNK), jnp.int32),
        pltpu.VMEM((2, CHUNK, H), jnp.float32),
        pltpu.VMEM((N_NODES,), jnp.int32),
        pltpu.VMEM((ZROWS, H), jnp.float32),
        pltpu.VMEM_SHARED((ACC_ROWS, H), jnp.float32),
        pltpu.VMEM_SHARED((ACC_ROWS, H), jnp.float32),
        pltpu.SemaphoreType.DMA((2,)),
        pltpu.SemaphoreType.DMA((2,)),
        pltpu.SemaphoreType.DMA((2,)),
    ],
)(_sc_body)


BM = 1000


def _tc_body(ep_ref, en_ref, x_ref, a_ref, b_ref, ws_ref, wp_ref, wn_ref,
             out_ref):
    t = jnp.concatenate([a_ref[0], a_ref[1]], axis=1)
    s = jnp.concatenate([b_ref[0], b_ref[1]], axis=1)
    hp = s + t
    ep = ep_ref[0, 0]
    en = en_ref[0, 0]
    hf = x_ref[...] + (1.0 + ep) * hp + (1.0 + en) * t
    o1 = jnp.dot(hf, ws_ref[...], preferred_element_type=jnp.float32)
    o2 = jnp.dot(hp, wp_ref[...], preferred_element_type=jnp.float32)
    o3 = jnp.dot(t, wn_ref[...], preferred_element_type=jnp.float32)
    out_ref[...] = jnp.maximum(jnp.concatenate([o1, o2, o3], axis=1), 0.0)


_tc_finish = pl.pallas_call(
    _tc_body,
    grid=(N_NODES // BM,),
    in_specs=[
        pl.BlockSpec(memory_space=pltpu.SMEM),
        pl.BlockSpec(memory_space=pltpu.SMEM),
        pl.BlockSpec((BM, D), lambda i: (i, 0)),
        pl.BlockSpec((NC, BM, H), lambda i: (0, i, 0)),
        pl.BlockSpec((NC, BM, H), lambda i: (0, i, 0)),
        pl.BlockSpec((D, D), lambda i: (0, 0)),
        pl.BlockSpec((D, D), lambda i: (0, 0)),
        pl.BlockSpec((D, D), lambda i: (0, 0)),
    ],
    out_specs=pl.BlockSpec((BM, 3 * D), lambda i: (i, 0)),
    out_shape=jax.ShapeDtypeStruct((N_NODES, 3 * D), jnp.float32),
)


@jax.jit
def kernel(x, edge_index, label, eps_pos, eps_neg,
           weight_self, weight_pos, weight_neg):
    src = edge_index[0].astype(jnp.int32)
    dst = edge_index[1].astype(jnp.int32)
    lbl = label.astype(jnp.int32)
    x2 = x.reshape(2 * N_NODES, H)

    pad = PAD_E - N_EDGES
    src_p = jnp.concatenate([src, jnp.zeros((pad,), jnp.int32)])
    dst_p = jnp.concatenate([dst, jnp.full((pad,), DUMMY, jnp.int32)])
    edges = jnp.stack([src_p.reshape(NS, CHUNKS, CHUNK),
                       dst_p.reshape(NS, CHUNKS, CHUNK)], axis=2)
    extra = jnp.broadcast_to(
        jnp.stack([jnp.zeros((CHUNK,), jnp.int32),
                   jnp.full((CHUNK,), DUMMY, jnp.int32)]),
        (NS, 2, 2, CHUNK))
    edges = jnp.concatenate([edges, extra], axis=1)

    t_half, s_half = _sc_scatter(x2, edges, lbl)

    return _tc_finish(eps_pos.reshape(1, 1), eps_neg.reshape(1, 1),
                      x, t_half, s_half,
                      weight_self, weight_pos, weight_neg)
